# Initial kernel scaffold; baseline (speedup 1.0000x reference)
import jax
import jax.numpy as jnp
from jax.experimental import pallas as pl


def kernel(q, k, v):
    raise NotImplementedError("write your pallas kernel here")


# speedup vs baseline: 1.4151x; 1.4151x over previous
import functools
import math

import jax
import jax.numpy as jnp
from jax import lax
from jax.experimental import pallas as pl
from jax.experimental.pallas import tpu as pltpu

BLK = 128
NB = 16
S = NB * BLK
D = 64
NSPECIAL = 4
CAT = S + S // 2 + S // 4 + S // 8 + BLK
MASK_RATIOS = ((1, 0.0, 0.05), (2, 0.05, 0.15), (4, 0.15, 0.25),
               (8, 0.25, 0.5), (0, 0.5, 1.0))
SIM_T2, SIM_T4, SIM_T8 = 0.75, 0.7, 0.7
import numpy as np

NEG = np.float32(-np.inf)
HI = lax.Precision.HIGHEST


def _pair_cos(a, b):
    num = (a * b).sum(-1)
    den = jnp.sqrt((a * a).sum(-1)) * jnp.sqrt((b * b).sum(-1)) + 1e-6
    return (num / den).mean(-1)


def _prep_body(q_ref, k_ref, v_ref, kcat_ref, vcat_ref, mask_ref):
    k = k_ref[0]
    v = v_ref[0]
    q = q_ref[0]

    k2 = k.reshape(S // 2, 2, D).mean(axis=1)
    k4 = k2.reshape(S // 4, 2, D).mean(axis=1)
    k8 = k4.reshape(S // 8, 2, D).mean(axis=1)
    kcat_ref[0] = jnp.concatenate(
        [k, k2, k4, k8, jnp.zeros((BLK, D), jnp.float32)], axis=0)
    v2 = v.reshape(S // 2, 2, D).mean(axis=1)
    v4 = v2.reshape(S // 4, 2, D).mean(axis=1)
    v8 = v4.reshape(S // 8, 2, D).mean(axis=1)
    vcat_ref[0] = jnp.concatenate(
        [v, v2, v4, v8, jnp.zeros((BLK, D), jnp.float32)], axis=0)

    qs = q.reshape(NB, 8, 16, D)[:, :, 0, :].mean(axis=1)
    ks = k.reshape(NB, 8, 16, D)[:, :, 0, :].mean(axis=1)
    scores = jnp.dot(qs, ks.T, precision=HI) * (1.0 / math.sqrt(D))
    row = lax.broadcasted_iota(jnp.int32, (NB, NB), 0)
    col = lax.broadcasted_iota(jnp.int32, (NB, NB), 1)
    scores = jnp.where(col <= row, scores, NEG)
    mx = jnp.max(scores, axis=-1, keepdims=True)
    e = jnp.exp(scores - mx)
    attn = e / jnp.sum(e, axis=-1, keepdims=True)

    ranks = (attn[:, :, None] > attn[:, None, :]).astype(jnp.int32).sum(axis=1)

    vi = (lax.broadcasted_iota(jnp.int32, (NB, 1), 0) + 1).astype(jnp.float32)
    maskv = jnp.zeros((NB, NB), jnp.int32)
    for value, sr, er in MASK_RATIOS:
        start = jnp.minimum((vi * sr).astype(jnp.int32), NB)
        end = jnp.minimum((vi * er).astype(jnp.int32), NB)
        in_range = (ranks >= start) & (ranks < end)
        maskv = jnp.where(in_range, jnp.int32(value), maskv)
    sp_col = col >= (NB - NSPECIAL)
    sp_row = row >= (NB - NSPECIAL)
    maskv = jnp.where(sp_col | sp_row, 1, maskv)
    maskv = jnp.where(col > row, 0, maskv)
    maskv = jnp.where(col == row, 1, maskv)
    maskv = jnp.where(col == 0, 1, maskv)

    p2 = k.reshape(NB, BLK // 2, 2, D)
    sim2 = _pair_cos(p2[:, :, 0, :], p2[:, :, 1, :])
    kk2 = p2.mean(axis=2)
    p4 = kk2.reshape(NB, BLK // 4, 2, D)
    sim4 = _pair_cos(p4[:, :, 0, :], p4[:, :, 1, :])
    kk4 = p4.mean(axis=2)
    p8 = kk4.reshape(NB, BLK // 8, 2, D)
    sim8 = _pair_cos(p8[:, :, 0, :], p8[:, :, 1, :])
    val = jnp.where(sim2 >= SIM_T2,
                    jnp.where(sim4 >= SIM_T4,
                              jnp.where(sim8 >= SIM_T8, 8, 4), 2), 1)
    maskv = jnp.minimum(maskv, val[None, :].astype(jnp.int32))
    mask_ref[0] = maskv


def _attn_body(mask_ref, q_ref, kcat_ref, vcat_ref, o_ref):
    i = pl.program_id(1)
    qb = q_ref[0]
    scale = np.float32(1.0 / math.sqrt(D))
    rowi = lax.broadcasted_iota(jnp.int32, (BLK, BLK), 0)
    coli = lax.broadcasted_iota(jnp.int32, (BLK, BLK), 1)

    def body(j, carry):
        m, l, acc = carry
        p = mask_ref[0, i, j]
        pe = jnp.maximum(p, 1)
        w = 128 // pe
        off = 4096 - 8192 // (2 * pe)
        start = off + j * w
        kblk = kcat_ref[0, pl.ds(start, BLK), :]
        vblk = vcat_ref[0, pl.ds(start, BLK), :]
        s = jnp.dot(qb, kblk.T, precision=HI) * scale
        valid = (coli < w) & (p > 0)
        c = jnp.clip(rowi + 1 - coli * pe, 0, pe)
        adj = jnp.where(c > 0,
                        jnp.log(c.astype(jnp.float32))
                        - jnp.log(pe.astype(jnp.float32)),
                        NEG)
        bias = jnp.where(valid, jnp.where(j == i, adj, 0.0), NEG)
        s = s + bias
        m_new = jnp.maximum(m, jnp.max(s, axis=1, keepdims=True))
        alpha = jnp.exp(m - m_new)
        pexp = jnp.exp(s - m_new)
        l_new = l * alpha + jnp.sum(pexp, axis=1, keepdims=True)
        acc_new = acc * alpha + jnp.dot(pexp, vblk, precision=HI)
        return m_new, l_new, acc_new

    m0 = jnp.full((BLK, 1), NEG, jnp.float32)
    l0 = jnp.zeros((BLK, 1), jnp.float32)
    a0 = jnp.zeros((BLK, D), jnp.float32)
    m, l, acc = lax.fori_loop(0, i + 1, body, (m0, l0, a0))
    o_ref[0] = acc / l


def _run(q3, k3, v3, interpret=False):
    H = q3.shape[0]
    kcat, vcat, mask = pl.pallas_call(
        _prep_body,
        grid=(H,),
        in_specs=[pl.BlockSpec((1, S, D), lambda h: (h, 0, 0))] * 3,
        out_specs=[
            pl.BlockSpec((1, CAT, D), lambda h: (h, 0, 0)),
            pl.BlockSpec((1, CAT, D), lambda h: (h, 0, 0)),
            pl.BlockSpec((1, NB, NB), lambda h: (h, 0, 0)),
        ],
        out_shape=[
            jax.ShapeDtypeStruct((H, CAT, D), jnp.float32),
            jax.ShapeDtypeStruct((H, CAT, D), jnp.float32),
            jax.ShapeDtypeStruct((H, NB, NB), jnp.int32),
        ],
        interpret=interpret,
    )(q3, k3, v3)

    out = pl.pallas_call(
        _attn_body,
        grid=(H, NB),
        in_specs=[
            pl.BlockSpec((1, NB, NB), lambda h, i: (h, 0, 0),
                         memory_space=pltpu.SMEM),
            pl.BlockSpec((1, BLK, D), lambda h, i: (h, i, 0)),
            pl.BlockSpec((1, CAT, D), lambda h, i: (h, 0, 0)),
            pl.BlockSpec((1, CAT, D), lambda h, i: (h, 0, 0)),
        ],
        out_specs=pl.BlockSpec((1, BLK, D), lambda h, i: (h, i, 0)),
        out_shape=jax.ShapeDtypeStruct((H, S, D), jnp.float32),
        interpret=interpret,
    )(mask, q3, kcat, vcat)
    return out


def kernel(q, k, v):
    B, H, s, d = q.shape
    assert s == S and d == D
    q3 = q.reshape(B * H, S, D)
    k3 = k.reshape(B * H, S, D)
    v3 = v.reshape(B * H, S, D)
    out = _run(q3, k3, v3)
    return out.reshape(B, H, S, D)


# speedup vs baseline: 1.4355x; 1.0144x over previous
import functools
import math

import jax
import jax.numpy as jnp
from jax import lax
from jax.experimental import pallas as pl
from jax.experimental.pallas import tpu as pltpu

BLK = 128
NB = 16
S = NB * BLK
D = 64
NSPECIAL = 4
CAT = S + S // 2 + S // 4 + S // 8 + BLK
MASK_RATIOS = ((1, 0.0, 0.05), (2, 0.05, 0.15), (4, 0.15, 0.25),
               (8, 0.25, 0.5), (0, 0.5, 1.0))
SIM_T2, SIM_T4, SIM_T8 = 0.75, 0.7, 0.7
import numpy as np

NEG = np.float32(-np.inf)
HI = lax.Precision.HIGHEST


def _pair_cos(a, b):
    num = (a * b).sum(-1)
    den = jnp.sqrt((a * a).sum(-1)) * jnp.sqrt((b * b).sum(-1)) + 1e-6
    return (num / den).mean(-1)


def _prep_body(q_ref, k_ref, v_ref, kcat_ref, vcat_ref, mask_ref):
    k = k_ref[0]
    v = v_ref[0]
    q = q_ref[0]

    k2 = k.reshape(S // 2, 2, D).mean(axis=1)
    k4 = k2.reshape(S // 4, 2, D).mean(axis=1)
    k8 = k4.reshape(S // 8, 2, D).mean(axis=1)
    kcat_ref[0] = jnp.concatenate(
        [k, k2, k4, k8, jnp.zeros((BLK, D), jnp.float32)], axis=0)
    v2 = v.reshape(S // 2, 2, D).mean(axis=1)
    v4 = v2.reshape(S // 4, 2, D).mean(axis=1)
    v8 = v4.reshape(S // 8, 2, D).mean(axis=1)
    vcat_ref[0] = jnp.concatenate(
        [v, v2, v4, v8, jnp.zeros((BLK, D), jnp.float32)], axis=0)

    qs = q.reshape(NB, 8, 16, D)[:, :, 0, :].mean(axis=1)
    ks = k.reshape(NB, 8, 16, D)[:, :, 0, :].mean(axis=1)
    scores = jnp.dot(qs, ks.T, precision=HI) * (1.0 / math.sqrt(D))
    row = lax.broadcasted_iota(jnp.int32, (NB, NB), 0)
    col = lax.broadcasted_iota(jnp.int32, (NB, NB), 1)
    scores = jnp.where(col <= row, scores, NEG)
    mx = jnp.max(scores, axis=-1, keepdims=True)
    e = jnp.exp(scores - mx)
    attn = e / jnp.sum(e, axis=-1, keepdims=True)

    ranks = (attn[:, :, None] > attn[:, None, :]).astype(jnp.int32).sum(axis=1)

    vi = (lax.broadcasted_iota(jnp.int32, (NB, 1), 0) + 1).astype(jnp.float32)
    maskv = jnp.zeros((NB, NB), jnp.int32)
    for value, sr, er in MASK_RATIOS:
        start = jnp.minimum((vi * sr).astype(jnp.int32), NB)
        end = jnp.minimum((vi * er).astype(jnp.int32), NB)
        in_range = (ranks >= start) & (ranks < end)
        maskv = jnp.where(in_range, jnp.int32(value), maskv)
    sp_col = col >= (NB - NSPECIAL)
    sp_row = row >= (NB - NSPECIAL)
    maskv = jnp.where(sp_col | sp_row, 1, maskv)
    maskv = jnp.where(col > row, 0, maskv)
    maskv = jnp.where(col == row, 1, maskv)
    maskv = jnp.where(col == 0, 1, maskv)

    p2 = k.reshape(NB, BLK // 2, 2, D)
    sim2 = _pair_cos(p2[:, :, 0, :], p2[:, :, 1, :])
    kk2 = p2.mean(axis=2)
    p4 = kk2.reshape(NB, BLK // 4, 2, D)
    sim4 = _pair_cos(p4[:, :, 0, :], p4[:, :, 1, :])
    kk4 = p4.mean(axis=2)
    p8 = kk4.reshape(NB, BLK // 8, 2, D)
    sim8 = _pair_cos(p8[:, :, 0, :], p8[:, :, 1, :])
    val = jnp.where(sim2 >= SIM_T2,
                    jnp.where(sim4 >= SIM_T4,
                              jnp.where(sim8 >= SIM_T8, 8, 4), 2), 1)
    maskv = jnp.minimum(maskv, val[None, :].astype(jnp.int32))
    mask_ref[0] = maskv


def _attn_body(mask_ref, q_ref, kcat_ref, vcat_ref, o_ref):
    i = pl.program_id(1)
    scale = np.float32(1.0 / math.sqrt(D))
    qs = q_ref[0] * scale
    rowi = lax.broadcasted_iota(jnp.int32, (BLK, BLK), 0)
    coli = lax.broadcasted_iota(jnp.int32, (BLK, BLK), 1)

    pd = mask_ref[0, i, i]
    wd = 128 // pd
    sd = (4096 - 8192 // (2 * pd)) + i * wd
    kblk = kcat_ref[0, pl.ds(sd, BLK), :]
    vblk = vcat_ref[0, pl.ds(sd, BLK), :]
    s = jnp.dot(qs, kblk.T, precision=HI)
    c = jnp.clip(rowi + 1 - coli * pd, 0, pd)
    bias = jnp.where((coli < wd) & (c > 0),
                     jnp.log(c.astype(jnp.float32))
                     - jnp.log(pd.astype(jnp.float32)),
                     NEG)
    s = s + bias
    m = jnp.max(s, axis=1, keepdims=True)
    e = jnp.exp(s - m)
    l = jnp.sum(e, axis=1, keepdims=True)
    acc = jnp.dot(e, vblk, precision=HI)

    def body(j, carry):
        m, l, acc = carry
        p = mask_ref[0, i, j]
        pe = jnp.maximum(p, 1)
        w = 128 // pe
        start = (4096 - 8192 // (2 * pe)) + j * w
        kblk = kcat_ref[0, pl.ds(start, BLK), :]
        vblk = vcat_ref[0, pl.ds(start, BLK), :]
        s = jnp.dot(qs, kblk.T, precision=HI)
        s = jnp.where((coli < w) & (p > 0), s, NEG)
        m_new = jnp.maximum(m, jnp.max(s, axis=1, keepdims=True))
        alpha = jnp.exp(m - m_new)
        pexp = jnp.exp(s - m_new)
        l_new = l * alpha + jnp.sum(pexp, axis=1, keepdims=True)
        acc_new = acc * alpha + jnp.dot(pexp, vblk, precision=HI)
        return m_new, l_new, acc_new

    m, l, acc = lax.fori_loop(0, i, body, (m, l, acc))
    o_ref[0] = acc / l


def _run(q3, k3, v3, interpret=False):
    H = q3.shape[0]
    kcat, vcat, mask = pl.pallas_call(
        _prep_body,
        grid=(H,),
        in_specs=[pl.BlockSpec((1, S, D), lambda h: (h, 0, 0))] * 3,
        out_specs=[
            pl.BlockSpec((1, CAT, D), lambda h: (h, 0, 0)),
            pl.BlockSpec((1, CAT, D), lambda h: (h, 0, 0)),
            pl.BlockSpec((1, NB, NB), lambda h: (h, 0, 0)),
        ],
        out_shape=[
            jax.ShapeDtypeStruct((H, CAT, D), jnp.float32),
            jax.ShapeDtypeStruct((H, CAT, D), jnp.float32),
            jax.ShapeDtypeStruct((H, NB, NB), jnp.int32),
        ],
        interpret=interpret,
    )(q3, k3, v3)

    out = pl.pallas_call(
        _attn_body,
        grid=(H, NB),
        in_specs=[
            pl.BlockSpec((1, NB, NB), lambda h, i: (h, 0, 0),
                         memory_space=pltpu.SMEM),
            pl.BlockSpec((1, BLK, D), lambda h, i: (h, i, 0)),
            pl.BlockSpec((1, CAT, D), lambda h, i: (h, 0, 0)),
            pl.BlockSpec((1, CAT, D), lambda h, i: (h, 0, 0)),
        ],
        out_specs=pl.BlockSpec((1, BLK, D), lambda h, i: (h, i, 0)),
        out_shape=jax.ShapeDtypeStruct((H, S, D), jnp.float32),
        interpret=interpret,
    )(mask, q3, kcat, vcat)
    return out


def kernel(q, k, v):
    B, H, s, d = q.shape
    assert s == S and d == D
    q3 = q.reshape(B * H, S, D)
    k3 = k.reshape(B * H, S, D)
    v3 = v.reshape(B * H, S, D)
    out = _run(q3, k3, v3)
    return out.reshape(B, H, S, D)


# speedup vs baseline: 2.5385x; 1.7684x over previous
import functools
import math

import jax
import jax.numpy as jnp
from jax import lax
from jax.experimental import pallas as pl
from jax.experimental.pallas import tpu as pltpu

BLK = 128
NB = 16
S = NB * BLK
D = 64
NSPECIAL = 4
CAT = S + S // 2 + S // 4 + S // 8 + BLK
MASK_RATIOS = ((1, 0.0, 0.05), (2, 0.05, 0.15), (4, 0.15, 0.25),
               (8, 0.25, 0.5), (0, 0.5, 1.0))
SIM_T2, SIM_T4, SIM_T8 = 0.75, 0.7, 0.7
import numpy as np

NEG = np.float32(-np.inf)
HI = lax.Precision.HIGHEST


def _pair_cos(a, b):
    num = (a * b).sum(-1)
    den = jnp.sqrt((a * a).sum(-1)) * jnp.sqrt((b * b).sum(-1)) + 1e-6
    return (num / den).mean(-1)


def _prep_body(q_ref, k_ref, v_ref, kcat_ref, vcat_ref, mask_ref):
    k = k_ref[0]
    v = v_ref[0]
    q = q_ref[0]

    k2 = k.reshape(S // 2, 2, D).mean(axis=1)
    k4 = k2.reshape(S // 4, 2, D).mean(axis=1)
    k8 = k4.reshape(S // 8, 2, D).mean(axis=1)
    kcat_ref[0] = jnp.concatenate(
        [k, k2, k4, k8, jnp.zeros((BLK, D), jnp.float32)], axis=0)
    v2 = v.reshape(S // 2, 2, D).mean(axis=1)
    v4 = v2.reshape(S // 4, 2, D).mean(axis=1)
    v8 = v4.reshape(S // 8, 2, D).mean(axis=1)
    vcat_ref[0] = jnp.concatenate(
        [v, v2, v4, v8, jnp.zeros((BLK, D), jnp.float32)], axis=0)

    qs = q.reshape(NB, 8, 16, D)[:, :, 0, :].mean(axis=1)
    ks = k.reshape(NB, 8, 16, D)[:, :, 0, :].mean(axis=1)
    scores = jnp.dot(qs, ks.T, precision=HI) * (1.0 / math.sqrt(D))
    row = lax.broadcasted_iota(jnp.int32, (NB, NB), 0)
    col = lax.broadcasted_iota(jnp.int32, (NB, NB), 1)
    scores = jnp.where(col <= row, scores, NEG)
    mx = jnp.max(scores, axis=-1, keepdims=True)
    e = jnp.exp(scores - mx)
    attn = e / jnp.sum(e, axis=-1, keepdims=True)

    ranks = (attn[:, :, None] > attn[:, None, :]).astype(jnp.int32).sum(axis=1)

    vi = (lax.broadcasted_iota(jnp.int32, (NB, 1), 0) + 1).astype(jnp.float32)
    maskv = jnp.zeros((NB, NB), jnp.int32)
    for value, sr, er in MASK_RATIOS:
        start = jnp.minimum((vi * sr).astype(jnp.int32), NB)
        end = jnp.minimum((vi * er).astype(jnp.int32), NB)
        in_range = (ranks >= start) & (ranks < end)
        maskv = jnp.where(in_range, jnp.int32(value), maskv)
    sp_col = col >= (NB - NSPECIAL)
    sp_row = row >= (NB - NSPECIAL)
    maskv = jnp.where(sp_col | sp_row, 1, maskv)
    maskv = jnp.where(col > row, 0, maskv)
    maskv = jnp.where(col == row, 1, maskv)
    maskv = jnp.where(col == 0, 1, maskv)

    p2 = k.reshape(NB, BLK // 2, 2, D)
    sim2 = _pair_cos(p2[:, :, 0, :], p2[:, :, 1, :])
    kk2 = p2.mean(axis=2)
    p4 = kk2.reshape(NB, BLK // 4, 2, D)
    sim4 = _pair_cos(p4[:, :, 0, :], p4[:, :, 1, :])
    kk4 = p4.mean(axis=2)
    p8 = kk4.reshape(NB, BLK // 8, 2, D)
    sim8 = _pair_cos(p8[:, :, 0, :], p8[:, :, 1, :])
    val = jnp.where(sim2 >= SIM_T2,
                    jnp.where(sim4 >= SIM_T4,
                              jnp.where(sim8 >= SIM_T8, 8, 4), 2), 1)
    maskv = jnp.minimum(maskv, val[None, :].astype(jnp.int32))
    mask_ref[0] = maskv


def _attn_body(mask_ref, q_ref, kcat_ref, vcat_ref, o_ref):
    i = pl.program_id(1)
    scale = np.float32(1.0 / math.sqrt(D))
    qs = q_ref[0] * scale
    rowi = lax.broadcasted_iota(jnp.int32, (BLK, BLK), 0)
    coli = lax.broadcasted_iota(jnp.int32, (BLK, BLK), 1)

    pd = mask_ref[0, i, i]
    wd = 128 // pd
    sd = (4096 - 8192 // (2 * pd)) + i * wd
    kblk = kcat_ref[0, pl.ds(sd, BLK), :]
    vblk = vcat_ref[0, pl.ds(sd, BLK), :]
    s = jnp.dot(qs, kblk.T, precision=HI)
    c = jnp.clip(rowi + 1 - coli * pd, 0, pd)
    bias = jnp.where((coli < wd) & (c > 0),
                     jnp.log(c.astype(jnp.float32))
                     - jnp.log(pd.astype(jnp.float32)),
                     NEG)
    s = s + bias
    m = jnp.max(s, axis=1, keepdims=True)
    e = jnp.exp(s - m)
    l = jnp.sum(e, axis=1, keepdims=True)
    acc = jnp.dot(e, vblk)

    def logits(j, p):
        pe = jnp.maximum(p, 1)
        w = 128 // pe
        start = (4096 - 8192 // (2 * pe)) + j * w
        kblk = kcat_ref[0, pl.ds(start, BLK), :]
        vblk = vcat_ref[0, pl.ds(start, BLK), :]
        sj = jnp.dot(qs, kblk.T, precision=HI)
        sj = jnp.where((coli < w) & (p > 0), sj, NEG)
        return sj, vblk

    def body(t, carry):
        m, l, acc = carry
        j0 = 2 * t
        j1 = 2 * t + 1
        p0 = mask_ref[0, i, j0]
        p1 = jnp.where(j1 < i, mask_ref[0, i, j1], 0)
        s0, v0 = logits(j0, p0)
        s1, v1 = logits(j1, p1)
        m_new = jnp.maximum(m, jnp.maximum(
            jnp.max(s0, axis=1, keepdims=True),
            jnp.max(s1, axis=1, keepdims=True)))
        alpha = jnp.exp(m - m_new)
        e0 = jnp.exp(s0 - m_new)
        e1 = jnp.exp(s1 - m_new)
        l_new = (l * alpha + jnp.sum(e0, axis=1, keepdims=True)
                 + jnp.sum(e1, axis=1, keepdims=True))
        acc_new = (acc * alpha + jnp.dot(e0, v0) + jnp.dot(e1, v1))
        return m_new, l_new, acc_new

    m, l, acc = lax.fori_loop(0, (i + 1) // 2, body, (m, l, acc))
    o_ref[0] = acc / l


def _run(q3, k3, v3, interpret=False):
    H = q3.shape[0]
    kcat, vcat, mask = pl.pallas_call(
        _prep_body,
        grid=(H,),
        in_specs=[pl.BlockSpec((1, S, D), lambda h: (h, 0, 0))] * 3,
        out_specs=[
            pl.BlockSpec((1, CAT, D), lambda h: (h, 0, 0)),
            pl.BlockSpec((1, CAT, D), lambda h: (h, 0, 0)),
            pl.BlockSpec((1, NB, NB), lambda h: (h, 0, 0)),
        ],
        out_shape=[
            jax.ShapeDtypeStruct((H, CAT, D), jnp.float32),
            jax.ShapeDtypeStruct((H, CAT, D), jnp.float32),
            jax.ShapeDtypeStruct((H, NB, NB), jnp.int32),
        ],
        interpret=interpret,
    )(q3, k3, v3)

    out = pl.pallas_call(
        _attn_body,
        grid=(H, NB),
        in_specs=[
            pl.BlockSpec((1, NB, NB), lambda h, i: (h, 0, 0),
                         memory_space=pltpu.SMEM),
            pl.BlockSpec((1, BLK, D), lambda h, i: (h, i, 0)),
            pl.BlockSpec((1, CAT, D), lambda h, i: (h, 0, 0)),
            pl.BlockSpec((1, CAT, D), lambda h, i: (h, 0, 0)),
        ],
        out_specs=pl.BlockSpec((1, BLK, D), lambda h, i: (h, i, 0)),
        out_shape=jax.ShapeDtypeStruct((H, S, D), jnp.float32),
        interpret=interpret,
    )(mask, q3, kcat, vcat)
    return out


def kernel(q, k, v):
    B, H, s, d = q.shape
    assert s == S and d == D
    q3 = q.reshape(B * H, S, D)
    k3 = k.reshape(B * H, S, D)
    v3 = v.reshape(B * H, S, D)
    out = _run(q3, k3, v3)
    return out.reshape(B, H, S, D)


# speedup vs baseline: 2.7043x; 1.0653x over previous
import functools
import math

import jax
import jax.numpy as jnp
from jax import lax
from jax.experimental import pallas as pl
from jax.experimental.pallas import tpu as pltpu

BLK = 128
NB = 16
S = NB * BLK
D = 64
NSPECIAL = 4
CAT = S + S // 2 + S // 4 + S // 8 + BLK
MASK_RATIOS = ((1, 0.0, 0.05), (2, 0.05, 0.15), (4, 0.15, 0.25),
               (8, 0.25, 0.5), (0, 0.5, 1.0))
SIM_T2, SIM_T4, SIM_T8 = 0.75, 0.7, 0.7
import numpy as np

NEG = np.float32(-np.inf)
HI = lax.Precision.HIGHEST


def _pair_cos(a, b):
    num = (a * b).sum(-1)
    den = jnp.sqrt((a * a).sum(-1)) * jnp.sqrt((b * b).sum(-1)) + 1e-6
    return (num / den).mean(-1)


def _prep_body(q_ref, k_ref, v_ref, kcat_ref, vcat_ref, mask_ref):
    k = k_ref[0]
    v = v_ref[0]
    q = q_ref[0]

    k2 = k.reshape(S // 2, 2, D).mean(axis=1)
    k4 = k2.reshape(S // 4, 2, D).mean(axis=1)
    k8 = k4.reshape(S // 8, 2, D).mean(axis=1)
    kcat_ref[0] = jnp.concatenate(
        [k, k2, k4, k8, jnp.zeros((BLK, D), jnp.float32)], axis=0)
    v2 = v.reshape(S // 2, 2, D).mean(axis=1)
    v4 = v2.reshape(S // 4, 2, D).mean(axis=1)
    v8 = v4.reshape(S // 8, 2, D).mean(axis=1)
    vcat_ref[0] = jnp.concatenate(
        [v, v2, v4, v8, jnp.zeros((BLK, D), jnp.float32)], axis=0)

    qs = q.reshape(NB, 8, 16, D)[:, :, 0, :].mean(axis=1)
    ks = k.reshape(NB, 8, 16, D)[:, :, 0, :].mean(axis=1)
    scores = jnp.dot(qs, ks.T, precision=HI) * (1.0 / math.sqrt(D))
    row = lax.broadcasted_iota(jnp.int32, (NB, NB), 0)
    col = lax.broadcasted_iota(jnp.int32, (NB, NB), 1)
    scores = jnp.where(col <= row, scores, NEG)
    mx = jnp.max(scores, axis=-1, keepdims=True)
    e = jnp.exp(scores - mx)
    attn = e / jnp.sum(e, axis=-1, keepdims=True)

    ranks = (attn[:, :, None] > attn[:, None, :]).astype(jnp.int32).sum(axis=1)

    vi = (lax.broadcasted_iota(jnp.int32, (NB, 1), 0) + 1).astype(jnp.float32)
    maskv = jnp.zeros((NB, NB), jnp.int32)
    for value, sr, er in MASK_RATIOS:
        start = jnp.minimum((vi * sr).astype(jnp.int32), NB)
        end = jnp.minimum((vi * er).astype(jnp.int32), NB)
        in_range = (ranks >= start) & (ranks < end)
        maskv = jnp.where(in_range, jnp.int32(value), maskv)
    sp_col = col >= (NB - NSPECIAL)
    sp_row = row >= (NB - NSPECIAL)
    maskv = jnp.where(sp_col | sp_row, 1, maskv)
    maskv = jnp.where(col > row, 0, maskv)
    maskv = jnp.where(col == row, 1, maskv)
    maskv = jnp.where(col == 0, 1, maskv)

    p2 = k.reshape(NB, BLK // 2, 2, D)
    sim2 = _pair_cos(p2[:, :, 0, :], p2[:, :, 1, :])
    kk2 = p2.mean(axis=2)
    p4 = kk2.reshape(NB, BLK // 4, 2, D)
    sim4 = _pair_cos(p4[:, :, 0, :], p4[:, :, 1, :])
    kk4 = p4.mean(axis=2)
    p8 = kk4.reshape(NB, BLK // 8, 2, D)
    sim8 = _pair_cos(p8[:, :, 0, :], p8[:, :, 1, :])
    val = jnp.where(sim2 >= SIM_T2,
                    jnp.where(sim4 >= SIM_T4,
                              jnp.where(sim8 >= SIM_T8, 8, 4), 2), 1)
    maskv = jnp.minimum(maskv, val[None, :].astype(jnp.int32))
    mask_ref[0] = maskv


def _attn_body(mask_ref, q_ref, kcat_ref, vcat_ref, o_ref):
    i = pl.program_id(1)
    scale = np.float32(1.0 / math.sqrt(D))
    qs = q_ref[0] * scale
    rowi = lax.broadcasted_iota(jnp.int32, (BLK, BLK), 0)
    coli = lax.broadcasted_iota(jnp.int32, (BLK, BLK), 1)

    pd = mask_ref[0, i, i]
    wd = 128 // pd
    sd = (4096 - 8192 // (2 * pd)) + i * wd
    kblk = kcat_ref[0, pl.ds(sd, BLK), :]
    vblk = vcat_ref[0, pl.ds(sd, BLK), :]
    s = jnp.dot(qs, kblk.T, precision=HI)
    c = jnp.clip(rowi + 1 - coli * pd, 0, pd)
    bias = jnp.where((coli < wd) & (c > 0),
                     jnp.log(c.astype(jnp.float32))
                     - jnp.log(pd.astype(jnp.float32)),
                     NEG)
    s = s + bias
    m = jnp.max(s, axis=1, keepdims=True)
    e = jnp.exp(s - m)
    l = jnp.sum(e, axis=1, keepdims=True)
    acc = jnp.dot(e, vblk)

    def logits(j, p):
        pe = jnp.maximum(p, 1)
        w = 128 // pe
        start = (4096 - 8192 // (2 * pe)) + j * w
        kblk = kcat_ref[0, pl.ds(start, BLK), :]
        vblk = vcat_ref[0, pl.ds(start, BLK), :]
        sj = jnp.dot(qs, kblk.T, precision=HI)
        sj = jnp.where((coli < w) & (p > 0), sj, NEG)
        return sj, vblk

    UNROLL = 4

    def body(t, carry):
        m, l, acc = carry
        js = [UNROLL * t + u for u in range(UNROLL)]
        ps = [mask_ref[0, i, js[0]]]
        ps += [jnp.where(js[u] < i, mask_ref[0, i, js[u]], 0)
               for u in range(1, UNROLL)]
        sv = [logits(js[u], ps[u]) for u in range(UNROLL)]
        mx = jnp.max(sv[0][0], axis=1, keepdims=True)
        for u in range(1, UNROLL):
            mx = jnp.maximum(mx, jnp.max(sv[u][0], axis=1, keepdims=True))
        m_new = jnp.maximum(m, mx)
        alpha = jnp.exp(m - m_new)
        es = [jnp.exp(sj - m_new) for sj, _ in sv]
        l_new = l * alpha
        for e in es:
            l_new = l_new + jnp.sum(e, axis=1, keepdims=True)
        acc_new = acc * alpha
        for e, (_, vb) in zip(es, sv):
            acc_new = acc_new + jnp.dot(e, vb)
        return m_new, l_new, acc_new

    m, l, acc = lax.fori_loop(0, (i + UNROLL - 1) // UNROLL, body, (m, l, acc))
    o_ref[0] = acc / l


def _run(q3, k3, v3, interpret=False):
    H = q3.shape[0]
    kcat, vcat, mask = pl.pallas_call(
        _prep_body,
        grid=(H,),
        in_specs=[pl.BlockSpec((1, S, D), lambda h: (h, 0, 0))] * 3,
        out_specs=[
            pl.BlockSpec((1, CAT, D), lambda h: (h, 0, 0)),
            pl.BlockSpec((1, CAT, D), lambda h: (h, 0, 0)),
            pl.BlockSpec((1, NB, NB), lambda h: (h, 0, 0)),
        ],
        out_shape=[
            jax.ShapeDtypeStruct((H, CAT, D), jnp.float32),
            jax.ShapeDtypeStruct((H, CAT, D), jnp.float32),
            jax.ShapeDtypeStruct((H, NB, NB), jnp.int32),
        ],
        interpret=interpret,
    )(q3, k3, v3)

    out = pl.pallas_call(
        _attn_body,
        grid=(H, NB),
        in_specs=[
            pl.BlockSpec((1, NB, NB), lambda h, i: (h, 0, 0),
                         memory_space=pltpu.SMEM),
            pl.BlockSpec((1, BLK, D), lambda h, i: (h, i, 0)),
            pl.BlockSpec((1, CAT, D), lambda h, i: (h, 0, 0)),
            pl.BlockSpec((1, CAT, D), lambda h, i: (h, 0, 0)),
        ],
        out_specs=pl.BlockSpec((1, BLK, D), lambda h, i: (h, i, 0)),
        out_shape=jax.ShapeDtypeStruct((H, S, D), jnp.float32),
        interpret=interpret,
    )(mask, q3, kcat, vcat)
    return out


def kernel(q, k, v):
    B, H, s, d = q.shape
    assert s == S and d == D
    q3 = q.reshape(B * H, S, D)
    k3 = k.reshape(B * H, S, D)
    v3 = v.reshape(B * H, S, D)
    out = _run(q3, k3, v3)
    return out.reshape(B, H, S, D)


# speedup vs baseline: 2.9105x; 1.0763x over previous
import functools
import math

import jax
import jax.numpy as jnp
from jax import lax
from jax.experimental import pallas as pl
from jax.experimental.pallas import tpu as pltpu

BLK = 128
NB = 16
S = NB * BLK
D = 64
NSPECIAL = 4
CAT = S + S // 2 + S // 4 + S // 8 + BLK
MASK_RATIOS = ((1, 0.0, 0.05), (2, 0.05, 0.15), (4, 0.15, 0.25),
               (8, 0.25, 0.5), (0, 0.5, 1.0))
SIM_T2, SIM_T4, SIM_T8 = 0.75, 0.7, 0.7
import numpy as np

NEG = np.float32(-np.inf)


def _make_adj_table():
    rows = np.arange(BLK)[:, None]
    cols = np.arange(BLK)[None, :]
    table = np.zeros((4, BLK, BLK), np.float32)
    for n, p in enumerate((1, 2, 4, 8)):
        c = np.clip(rows + 1 - cols * p, 0, p).astype(np.float64)
        with np.errstate(divide="ignore"):
            table[n] = np.where(c > 0, np.log(c) - math.log(p),
                                -np.inf).astype(np.float32)
    return table


_ADJ_TABLE = _make_adj_table()
HI = lax.Precision.HIGHEST


def _pair_cos(a, b):
    num = (a * b).sum(-1)
    den = jnp.sqrt((a * a).sum(-1)) * jnp.sqrt((b * b).sum(-1)) + 1e-6
    return (num / den).mean(-1)


def _prep_body(q_ref, k_ref, v_ref, kcat_ref, vcat_ref, mask_ref):
    k = k_ref[0]
    v = v_ref[0]
    q = q_ref[0]

    k2 = k.reshape(S // 2, 2, D).mean(axis=1)
    k4 = k2.reshape(S // 4, 2, D).mean(axis=1)
    k8 = k4.reshape(S // 8, 2, D).mean(axis=1)
    kcat_ref[0] = jnp.concatenate(
        [k, k2, k4, k8, jnp.zeros((BLK, D), jnp.float32)], axis=0)
    v2 = v.reshape(S // 2, 2, D).mean(axis=1)
    v4 = v2.reshape(S // 4, 2, D).mean(axis=1)
    v8 = v4.reshape(S // 8, 2, D).mean(axis=1)
    vcat_ref[0] = jnp.concatenate(
        [v, v2, v4, v8, jnp.zeros((BLK, D), jnp.float32)], axis=0)

    qs = q.reshape(NB, 8, 16, D)[:, :, 0, :].mean(axis=1)
    ks = k.reshape(NB, 8, 16, D)[:, :, 0, :].mean(axis=1)
    scores = jnp.dot(qs, ks.T, precision=HI) * (1.0 / math.sqrt(D))
    row = lax.broadcasted_iota(jnp.int32, (NB, NB), 0)
    col = lax.broadcasted_iota(jnp.int32, (NB, NB), 1)
    scores = jnp.where(col <= row, scores, NEG)
    mx = jnp.max(scores, axis=-1, keepdims=True)
    e = jnp.exp(scores - mx)
    attn = e / jnp.sum(e, axis=-1, keepdims=True)

    ranks = (attn[:, :, None] > attn[:, None, :]).astype(jnp.int32).sum(axis=1)

    vi = (lax.broadcasted_iota(jnp.int32, (NB, 1), 0) + 1).astype(jnp.float32)
    maskv = jnp.zeros((NB, NB), jnp.int32)
    for value, sr, er in MASK_RATIOS:
        start = jnp.minimum((vi * sr).astype(jnp.int32), NB)
        end = jnp.minimum((vi * er).astype(jnp.int32), NB)
        in_range = (ranks >= start) & (ranks < end)
        maskv = jnp.where(in_range, jnp.int32(value), maskv)
    sp_col = col >= (NB - NSPECIAL)
    sp_row = row >= (NB - NSPECIAL)
    maskv = jnp.where(sp_col | sp_row, 1, maskv)
    maskv = jnp.where(col > row, 0, maskv)
    maskv = jnp.where(col == row, 1, maskv)
    maskv = jnp.where(col == 0, 1, maskv)

    p2 = k.reshape(NB, BLK // 2, 2, D)
    sim2 = _pair_cos(p2[:, :, 0, :], p2[:, :, 1, :])
    kk2 = p2.mean(axis=2)
    p4 = kk2.reshape(NB, BLK // 4, 2, D)
    sim4 = _pair_cos(p4[:, :, 0, :], p4[:, :, 1, :])
    kk4 = p4.mean(axis=2)
    p8 = kk4.reshape(NB, BLK // 8, 2, D)
    sim8 = _pair_cos(p8[:, :, 0, :], p8[:, :, 1, :])
    val = jnp.where(sim2 >= SIM_T2,
                    jnp.where(sim4 >= SIM_T4,
                              jnp.where(sim8 >= SIM_T8, 8, 4), 2), 1)
    maskv = jnp.minimum(maskv, val[None, :].astype(jnp.int32))
    mask_ref[0] = maskv


def _attn_body(mask_ref, adj_ref, q_ref, kcat_ref, vcat_ref, o_ref):
    i = pl.program_id(1)
    scale = np.float32(1.0 / math.sqrt(D))
    qs = q_ref[0] * scale
    coli = lax.broadcasted_iota(jnp.int32, (BLK, BLK), 1)

    pd = mask_ref[0, i, i]
    idx = ((pd > 1).astype(jnp.int32) + (pd > 2).astype(jnp.int32)
           + (pd > 4).astype(jnp.int32))
    adj = adj_ref[idx]

    def logits(j, p):
        pe = jnp.maximum(p, 1)
        w = 128 // pe
        start = (4096 - 8192 // (2 * pe)) + j * w
        kblk = kcat_ref[0, pl.ds(start, BLK), :]
        vblk = vcat_ref[0, pl.ds(start, BLK), :]
        sj = jnp.dot(qs, kblk.T, precision=HI)
        offdiag = jnp.where((coli < w) & (p > 0), 0.0, NEG)
        sj = sj + jnp.where(j == i, adj, offdiag)
        return sj, vblk

    UNROLL = 4

    def body(t, carry):
        m, l, acc = carry
        js = [UNROLL * t + u for u in range(UNROLL)]
        ps = [mask_ref[0, i, js[0]]]
        ps += [jnp.where(js[u] <= i,
                         mask_ref[0, i, jnp.minimum(js[u], NB - 1)], 0)
               for u in range(1, UNROLL)]
        sv = [logits(js[u], ps[u]) for u in range(UNROLL)]
        mx = jnp.max(sv[0][0], axis=1, keepdims=True)
        for u in range(1, UNROLL):
            mx = jnp.maximum(mx, jnp.max(sv[u][0], axis=1, keepdims=True))
        m_new = jnp.maximum(m, mx)
        alpha = jnp.exp(m - m_new)
        es = [jnp.exp(sj - m_new) for sj, _ in sv]
        l_new = l * alpha
        for e in es:
            l_new = l_new + jnp.sum(e, axis=1, keepdims=True)
        acc_new = acc * alpha
        for e, (_, vb) in zip(es, sv):
            acc_new = acc_new + jnp.dot(e, vb)
        return m_new, l_new, acc_new

    m0 = jnp.full((BLK, 1), NEG, jnp.float32)
    l0 = jnp.zeros((BLK, 1), jnp.float32)
    a0 = jnp.zeros((BLK, D), jnp.float32)
    m, l, acc = lax.fori_loop(0, (i + UNROLL) // UNROLL, body, (m0, l0, a0))
    o_ref[0] = acc / l


def _run(q3, k3, v3, interpret=False):
    H = q3.shape[0]
    kcat, vcat, mask = pl.pallas_call(
        _prep_body,
        grid=(H,),
        in_specs=[pl.BlockSpec((1, S, D), lambda h: (h, 0, 0))] * 3,
        out_specs=[
            pl.BlockSpec((1, CAT, D), lambda h: (h, 0, 0)),
            pl.BlockSpec((1, CAT, D), lambda h: (h, 0, 0)),
            pl.BlockSpec((1, NB, NB), lambda h: (h, 0, 0)),
        ],
        out_shape=[
            jax.ShapeDtypeStruct((H, CAT, D), jnp.float32),
            jax.ShapeDtypeStruct((H, CAT, D), jnp.float32),
            jax.ShapeDtypeStruct((H, NB, NB), jnp.int32),
        ],
        interpret=interpret,
    )(q3, k3, v3)

    out = pl.pallas_call(
        _attn_body,
        grid=(H, NB),
        in_specs=[
            pl.BlockSpec((1, NB, NB), lambda h, i: (h, 0, 0),
                         memory_space=pltpu.SMEM),
            pl.BlockSpec((4, BLK, BLK), lambda h, i: (0, 0, 0)),
            pl.BlockSpec((1, BLK, D), lambda h, i: (h, i, 0)),
            pl.BlockSpec((1, CAT, D), lambda h, i: (h, 0, 0)),
            pl.BlockSpec((1, CAT, D), lambda h, i: (h, 0, 0)),
        ],
        out_specs=pl.BlockSpec((1, BLK, D), lambda h, i: (h, i, 0)),
        out_shape=jax.ShapeDtypeStruct((H, S, D), jnp.float32),
        interpret=interpret,
    )(mask, _ADJ_TABLE, q3, kcat, vcat)
    return out


def kernel(q, k, v):
    B, H, s, d = q.shape
    assert s == S and d == D
    q3 = q.reshape(B * H, S, D)
    k3 = k.reshape(B * H, S, D)
    v3 = v.reshape(B * H, S, D)
    out = _run(q3, k3, v3)
    return out.reshape(B, H, S, D)
